# baseline (device time: 58981 ns/iter reference)
import jax
import jax.numpy as jnp
from jax import lax
from jax.experimental import pallas as pl
from jax.experimental.pallas import tpu as pltpu

N_DEV = 8
B_LOC = 2
B = 16
SQ = 128
D = 512
HQ_LOC = 4
DH = 64
HD_LOC = HQ_LOC * DH
R_LOC = B_LOC * SQ


def _rope_tables():
    lane = lax.broadcasted_iota(jnp.int32, (SQ, HD_LOC), 1)
    pos = lax.broadcasted_iota(jnp.int32, (SQ, HD_LOC), 0).astype(jnp.float32)
    k = (lane % DH) // 2
    inv = jnp.exp(k.astype(jnp.float32) * (-2.0 / DH * jnp.log(10000.0)))
    ang = pos * inv
    return jnp.cos(ang), jnp.sin(ang)


def _rotate_half(t2):
    r_up = pltpu.roll(t2, 1, 1)
    r_dn = pltpu.roll(t2, t2.shape[1] - 1, 1)
    lane = lax.broadcasted_iota(jnp.int32, t2.shape, 1)
    return jnp.where(lane % 2 == 0, -r_dn, r_up)


def kernel(x, Wq, Wk, Wv, Wo):
    def body(x_ref, wq_ref, wk_ref, wv_ref, wo_ref, out_ref,
             xg, partial, stg, ag_send, ag_recv, rs_send, rs_recv):
        my = lax.axis_index("i")
        left = (my + N_DEV - 1) % N_DEV
        right = (my + 1) % N_DEV

        barrier = pltpu.get_barrier_semaphore()
        for nbr in (left, right):
            pl.semaphore_signal(
                barrier, inc=1, device_id=(nbr,),
                device_id_type=pl.DeviceIdType.MESH,
            )
        pl.semaphore_wait(barrier, 2)

        wq = wq_ref[...].astype(jnp.bfloat16)
        wk = wk_ref[...].astype(jnp.bfloat16)
        wv = wv_ref[...].astype(jnp.bfloat16)
        wo = wo_ref[...].astype(jnp.bfloat16)
        cos, sin = _rope_tables()
        cos3, sin3 = cos[None], sin[None]

        def chunk_partial(xc):
            def proj_rope(w):
                t2 = jnp.dot(xc, w, preferred_element_type=jnp.float32)
                tr2 = _rotate_half(t2)
                t3 = t2.reshape(B_LOC, SQ, HD_LOC)
                tr3 = tr2.reshape(B_LOC, SQ, HD_LOC)
                return (t3 * cos3 + tr3 * sin3).astype(jnp.bfloat16)

            q3 = proj_rope(wq)
            k3 = proj_rope(wk)
            v3 = (
                jnp.dot(xc, wv, preferred_element_type=jnp.float32)
                .reshape(B_LOC, SQ, HD_LOC)
                .astype(jnp.bfloat16)
            )
            rows = []
            for b in range(B_LOC):
                ctxs = []
                for h in range(HQ_LOC):
                    sl = slice(h * DH, (h + 1) * DH)
                    sc = lax.dot_general(
                        q3[b, :, sl], k3[b, :, sl],
                        (((1,), (1,)), ((), ())),
                        preferred_element_type=jnp.float32,
                    ) * 0.125
                    m = jnp.max(sc, axis=-1, keepdims=True)
                    w = jnp.exp(sc - m)
                    w = w / jnp.sum(w, axis=-1, keepdims=True)
                    ctxs.append(jnp.dot(
                        w.astype(jnp.bfloat16), v3[b, :, sl],
                        preferred_element_type=jnp.float32,
                    ))
                rows.append(jnp.concatenate(ctxs, axis=1))
            ctx2 = jnp.stack(rows).reshape(R_LOC, HD_LOC).astype(jnp.bfloat16)
            part = jnp.dot(ctx2, wo, preferred_element_type=jnp.float32)
            return part.reshape(B_LOC, SQ, D)

        def ag_hop(s):
            c = (my - s) % N_DEV
            rdma = pltpu.make_async_remote_copy(
                src_ref=xg.at[c], dst_ref=xg.at[c],
                send_sem=ag_send.at[s], recv_sem=ag_recv.at[s],
                device_id=(right,), device_id_type=pl.DeviceIdType.MESH,
            )
            rdma.start()
            return rdma

        xg[my] = x_ref[...].astype(jnp.bfloat16)
        ag = [ag_hop(0)]
        partial[my] = chunk_partial(
            x_ref[...].reshape(R_LOC, D).astype(jnp.bfloat16)
        ).astype(jnp.bfloat16)

        rs = []
        pending = None
        for r in range(N_DEV):
            if r < N_DEV - 1:
                ag[r].wait_recv()
                if r < N_DEV - 2:
                    ag.append(ag_hop(r + 1))
            if r >= 1:
                s = r - 1
                c = (my - 1 - s) % N_DEV
                part = pending
                if s > 0:
                    rs[s - 1].wait_recv()
                    part = part + stg[s - 1].astype(jnp.float32)
                partial[c] = part.astype(jnp.bfloat16)
                rdma = pltpu.make_async_remote_copy(
                    src_ref=partial.at[c], dst_ref=stg.at[s],
                    send_sem=rs_send.at[s], recv_sem=rs_recv.at[s],
                    device_id=(right,), device_id_type=pl.DeviceIdType.MESH,
                )
                rdma.start()
                rs.append(rdma)
            if r < N_DEV - 1:
                c2 = (my - 1 - r) % N_DEV
                pending = chunk_partial(xg[pl.ds(c2, 1)].reshape(R_LOC, D))

        rs[N_DEV - 2].wait_recv()
        out_ref[...] = (
            partial[pl.ds(my, 1)].reshape(B_LOC, SQ, D).astype(jnp.float32)
            + stg[N_DEV - 2].astype(jnp.float32)
        )

        for r in ag + rs:
            r.wait_send()

    out_shape = jax.ShapeDtypeStruct((B_LOC, SQ, D), jnp.float32)
    return pl.pallas_call(
        body,
        out_shape=out_shape,
        in_specs=[pl.BlockSpec(memory_space=pltpu.VMEM)] * 5,
        out_specs=pl.BlockSpec(memory_space=pltpu.VMEM),
        scratch_shapes=[
            pltpu.VMEM((N_DEV, B_LOC, SQ, D), jnp.bfloat16),
            pltpu.VMEM((N_DEV, B_LOC, SQ, D), jnp.bfloat16),
            pltpu.VMEM((N_DEV - 1, B_LOC, SQ, D), jnp.bfloat16),
            pltpu.SemaphoreType.DMA((N_DEV - 1,)),
            pltpu.SemaphoreType.DMA((N_DEV - 1,)),
            pltpu.SemaphoreType.DMA((N_DEV - 1,)),
            pltpu.SemaphoreType.DMA((N_DEV - 1,)),
        ],
        compiler_params=pltpu.CompilerParams(collective_id=0),
    )(x, Wq, Wk, Wv, Wo)


# device time: 54299 ns/iter; 1.0862x vs baseline; 1.0862x over previous
import jax
import jax.numpy as jnp
from jax import lax
from jax.experimental import pallas as pl
from jax.experimental.pallas import tpu as pltpu

N_DEV = 8
B_LOC = 2
B = 16
SQ = 128
D = 512
HQ_LOC = 4
DH = 64
HD_LOC = HQ_LOC * DH
R_LOC = B_LOC * SQ
NR = 4
NL = 3


def _rope_tables():
    lane = lax.broadcasted_iota(jnp.int32, (SQ, HD_LOC), 1)
    pos = lax.broadcasted_iota(jnp.int32, (SQ, HD_LOC), 0).astype(jnp.float32)
    k = (lane % DH) // 2
    inv = jnp.exp(k.astype(jnp.float32) * (-2.0 / DH * jnp.log(10000.0)))
    ang = pos * inv
    return jnp.cos(ang), jnp.sin(ang)


def _rotate_half(t2):
    r_up = pltpu.roll(t2, 1, 1)
    r_dn = pltpu.roll(t2, t2.shape[1] - 1, 1)
    lane = lax.broadcasted_iota(jnp.int32, t2.shape, 1)
    return jnp.where(lane % 2 == 0, -r_dn, r_up)


def kernel(x, Wq, Wk, Wv, Wo):
    def body(x_ref, wq_ref, wk_ref, wv_ref, wo_ref, out_ref,
             xg, partial, stg_r, stg_l,
             agr_send, agr_recv, agl_send, agl_recv,
             rsr_send, rsr_recv, rsl_send, rsl_recv):
        my = lax.axis_index("i")
        left = (my + N_DEV - 1) % N_DEV
        right = (my + 1) % N_DEV

        barrier = pltpu.get_barrier_semaphore()
        for nbr in (left, right):
            pl.semaphore_signal(
                barrier, inc=1, device_id=(nbr,),
                device_id_type=pl.DeviceIdType.MESH,
            )
        pl.semaphore_wait(barrier, 2)

        wq = wq_ref[...].astype(jnp.bfloat16)
        wk = wk_ref[...].astype(jnp.bfloat16)
        wv = wv_ref[...].astype(jnp.bfloat16)
        wo = wo_ref[...].astype(jnp.bfloat16)
        cos, sin = _rope_tables()
        cos3, sin3 = cos[None], sin[None]

        def chunk_partial(xc):
            def proj_rope(w):
                t2 = jnp.dot(xc, w, preferred_element_type=jnp.float32)
                tr2 = _rotate_half(t2)
                t3 = t2.reshape(B_LOC, SQ, HD_LOC)
                tr3 = tr2.reshape(B_LOC, SQ, HD_LOC)
                return (t3 * cos3 + tr3 * sin3).astype(jnp.bfloat16)

            q3 = proj_rope(wq)
            k3 = proj_rope(wk)
            v3 = (
                jnp.dot(xc, wv, preferred_element_type=jnp.float32)
                .reshape(B_LOC, SQ, HD_LOC)
                .astype(jnp.bfloat16)
            )
            rows = []
            for b in range(B_LOC):
                ctxs = []
                for h in range(HQ_LOC):
                    sl = slice(h * DH, (h + 1) * DH)
                    sc = lax.dot_general(
                        q3[b, :, sl], k3[b, :, sl],
                        (((1,), (1,)), ((), ())),
                        preferred_element_type=jnp.float32,
                    ) * 0.125
                    m = jnp.max(sc, axis=-1, keepdims=True)
                    w = jnp.exp(sc - m)
                    w = w / jnp.sum(w, axis=-1, keepdims=True)
                    ctxs.append(jnp.dot(
                        w.astype(jnp.bfloat16), v3[b, :, sl],
                        preferred_element_type=jnp.float32,
                    ))
                rows.append(jnp.concatenate(ctxs, axis=1))
            ctx2 = jnp.stack(rows).reshape(R_LOC, HD_LOC).astype(jnp.bfloat16)
            part = jnp.dot(ctx2, wo, preferred_element_type=jnp.float32)
            return part.reshape(B_LOC, SQ, D)

        def copy(src, dst, ssem, rsem, dev):
            rdma = pltpu.make_async_remote_copy(
                src_ref=src, dst_ref=dst, send_sem=ssem, recv_sem=rsem,
                device_id=(dev,), device_id_type=pl.DeviceIdType.MESH,
            )
            rdma.start()
            return rdma

        def ag_r(r):
            c = (my - r) % N_DEV
            return copy(xg.at[c], xg.at[c], agr_send.at[r], agr_recv.at[r],
                        right)

        def ag_l(r):
            c = (my + r) % N_DEV
            return copy(xg.at[c], xg.at[c], agl_send.at[r], agl_recv.at[r],
                        left)

        def rs_r(s):
            c = (my + NR - s) % N_DEV
            return copy(partial.at[c], stg_r.at[s], rsr_send.at[s],
                        rsr_recv.at[s], right)

        def rs_l(s):
            c = (my - NL + s) % N_DEV
            return copy(partial.at[c], stg_l.at[s], rsl_send.at[s],
                        rsl_recv.at[s], left)

        def accum(stg_slot, c):
            acc = (
                stg_slot.astype(jnp.float32)
                + partial[pl.ds(c, 1)].reshape(B_LOC, SQ, D).astype(jnp.float32)
            )
            partial[c] = acc.astype(jnp.bfloat16)

        xg[my] = x_ref[...].astype(jnp.bfloat16)
        agr = [ag_r(0)]
        agl = [ag_l(0)]
        partial[my] = chunk_partial(
            x_ref[...].reshape(R_LOC, D).astype(jnp.bfloat16)
        ).astype(jnp.bfloat16)

        rsl = []
        rsr = []
        for r in range(NR):
            agr[r].wait_recv()
            if r < NR - 1:
                agr.append(ag_r(r + 1))
            if r < NL:
                agl[r].wait_recv()
                if r < NL - 1:
                    agl.append(ag_l(r + 1))
            cr = (my - 1 - r) % N_DEV
            partial[cr] = chunk_partial(
                xg[pl.ds(cr, 1)].reshape(R_LOC, D)
            ).astype(jnp.bfloat16)
            if r == NL - 1:
                rsl.append(rs_l(0))
            if r < NL:
                cl = (my + 1 + r) % N_DEV
                partial[cl] = chunk_partial(
                    xg[pl.ds(cl, 1)].reshape(R_LOC, D)
                ).astype(jnp.bfloat16)
        rsr.append(rs_r(0))

        for s in range(NL):
            rsl[s].wait_recv()
            if s < NL - 1:
                accum(stg_l[s], (my - 2 + s) % N_DEV)
                rsl.append(rs_l(s + 1))
            rsr[s].wait_recv()
            accum(stg_r[s], (my + NL - s) % N_DEV)
            rsr.append(rs_r(s + 1))
        rsr[NR - 1].wait_recv()

        out_ref[...] = (
            partial[pl.ds(my, 1)].reshape(B_LOC, SQ, D).astype(jnp.float32)
            + stg_r[NR - 1].astype(jnp.float32)
            + stg_l[NL - 1].astype(jnp.float32)
        )

        for rdma in agr + agl + rsr + rsl:
            rdma.wait_send()

    out_shape = jax.ShapeDtypeStruct((B_LOC, SQ, D), jnp.float32)
    return pl.pallas_call(
        body,
        out_shape=out_shape,
        in_specs=[pl.BlockSpec(memory_space=pltpu.VMEM)] * 5,
        out_specs=pl.BlockSpec(memory_space=pltpu.VMEM),
        scratch_shapes=[
            pltpu.VMEM((N_DEV, B_LOC, SQ, D), jnp.bfloat16),
            pltpu.VMEM((N_DEV, B_LOC, SQ, D), jnp.bfloat16),
            pltpu.VMEM((NR, B_LOC, SQ, D), jnp.bfloat16),
            pltpu.VMEM((NL, B_LOC, SQ, D), jnp.bfloat16),
            pltpu.SemaphoreType.DMA((NR,)),
            pltpu.SemaphoreType.DMA((NR,)),
            pltpu.SemaphoreType.DMA((NL,)),
            pltpu.SemaphoreType.DMA((NL,)),
            pltpu.SemaphoreType.DMA((NR,)),
            pltpu.SemaphoreType.DMA((NR,)),
            pltpu.SemaphoreType.DMA((NL,)),
            pltpu.SemaphoreType.DMA((NL,)),
        ],
        compiler_params=pltpu.CompilerParams(collective_id=0),
    )(x, Wq, Wk, Wv, Wo)


# device time: 42613 ns/iter; 1.3841x vs baseline; 1.2742x over previous
import jax
import jax.numpy as jnp
from jax import lax
from jax.experimental import pallas as pl
from jax.experimental.pallas import tpu as pltpu

N_DEV = 8
B_LOC = 2
B = 16
SQ = 128
D = 512
HQ_LOC = 4
DH = 64
HD_LOC = HQ_LOC * DH
R_LOC = B_LOC * SQ

_K_ORDER = (1, 7, 2, 6, 3, 5, 4)


def _rope_tables():
    lane = lax.broadcasted_iota(jnp.int32, (SQ, HD_LOC), 1)
    pos = lax.broadcasted_iota(jnp.int32, (SQ, HD_LOC), 0).astype(jnp.float32)
    k = (lane % DH) // 2
    inv = jnp.exp(k.astype(jnp.float32) * (-2.0 / DH * jnp.log(10000.0)))
    ang = pos * inv
    return jnp.cos(ang), jnp.sin(ang)


def _rotate_half(t2):
    r_up = pltpu.roll(t2, 1, 1)
    r_dn = pltpu.roll(t2, t2.shape[1] - 1, 1)
    lane = lax.broadcasted_iota(jnp.int32, t2.shape, 1)
    return jnp.where(lane % 2 == 0, -r_dn, r_up)


def kernel(x, Wq, Wk, Wv, Wo):
    def body(x_ref, wq_ref, wk_ref, wv_ref, wo_ref, out_ref,
             xg, partial, stage,
             ag_send, ag_recv, rs_send, rs_recv):
        my = lax.axis_index("i")

        barrier = pltpu.get_barrier_semaphore()
        for k in range(1, N_DEV):
            pl.semaphore_signal(
                barrier, inc=1, device_id=((my + k) % N_DEV,),
                device_id_type=pl.DeviceIdType.MESH,
            )
        pl.semaphore_wait(barrier, N_DEV - 1)

        wq = wq_ref[...].astype(jnp.bfloat16)
        wk = wk_ref[...].astype(jnp.bfloat16)
        wv = wv_ref[...].astype(jnp.bfloat16)
        wo = wo_ref[...].astype(jnp.bfloat16)
        cos, sin = _rope_tables()
        cos3, sin3 = cos[None], sin[None]

        def chunk_partial(xc):
            def proj_rope(w):
                t2 = jnp.dot(xc, w, preferred_element_type=jnp.float32)
                tr2 = _rotate_half(t2)
                t3 = t2.reshape(B_LOC, SQ, HD_LOC)
                tr3 = tr2.reshape(B_LOC, SQ, HD_LOC)
                return (t3 * cos3 + tr3 * sin3).astype(jnp.bfloat16)

            q3 = proj_rope(wq)
            k3 = proj_rope(wk)
            v3 = (
                jnp.dot(xc, wv, preferred_element_type=jnp.float32)
                .reshape(B_LOC, SQ, HD_LOC)
                .astype(jnp.bfloat16)
            )
            rows = []
            for b in range(B_LOC):
                ctxs = []
                for h in range(HQ_LOC):
                    sl = slice(h * DH, (h + 1) * DH)
                    sc = lax.dot_general(
                        q3[b, :, sl], k3[b, :, sl],
                        (((1,), (1,)), ((), ())),
                        preferred_element_type=jnp.float32,
                    ) * 0.125
                    m = jnp.max(sc, axis=-1, keepdims=True)
                    w = jnp.exp(sc - m)
                    w = w / jnp.sum(w, axis=-1, keepdims=True)
                    ctxs.append(jnp.dot(
                        w.astype(jnp.bfloat16), v3[b, :, sl],
                        preferred_element_type=jnp.float32,
                    ))
                rows.append(jnp.concatenate(ctxs, axis=1))
            ctx2 = jnp.stack(rows).reshape(R_LOC, HD_LOC).astype(jnp.bfloat16)
            part = jnp.dot(ctx2, wo, preferred_element_type=jnp.float32)
            return part.reshape(B_LOC, SQ, D)

        xg[my] = x_ref[...].astype(jnp.bfloat16)
        ag = []
        for k in _K_ORDER:
            rdma = pltpu.make_async_remote_copy(
                src_ref=xg.at[my], dst_ref=xg.at[my],
                send_sem=ag_send.at[k - 1], recv_sem=ag_recv.at[k - 1],
                device_id=((my + k) % N_DEV,),
                device_id_type=pl.DeviceIdType.MESH,
            )
            rdma.start()
            ag.append(rdma)

        rs = []
        for i, k in enumerate(_K_ORDER):
            ag[i].wait_recv()
            c = (my - k) % N_DEV
            partial[c] = chunk_partial(
                xg[pl.ds(c, 1)].reshape(R_LOC, D)
            ).astype(jnp.bfloat16)
            rdma = pltpu.make_async_remote_copy(
                src_ref=partial.at[c], dst_ref=stage.at[k - 1],
                send_sem=rs_send.at[k - 1], recv_sem=rs_recv.at[k - 1],
                device_id=(c,),
                device_id_type=pl.DeviceIdType.MESH,
            )
            rdma.start()
            rs.append(rdma)

        acc = chunk_partial(x_ref[...].reshape(R_LOC, D).astype(jnp.bfloat16))
        for i, k in enumerate(_K_ORDER):
            rs[i].wait_recv()
            acc = acc + stage[k - 1].astype(jnp.float32)
        out_ref[...] = acc

        for rdma in ag + rs:
            rdma.wait_send()

    out_shape = jax.ShapeDtypeStruct((B_LOC, SQ, D), jnp.float32)
    return pl.pallas_call(
        body,
        out_shape=out_shape,
        in_specs=[pl.BlockSpec(memory_space=pltpu.VMEM)] * 5,
        out_specs=pl.BlockSpec(memory_space=pltpu.VMEM),
        scratch_shapes=[
            pltpu.VMEM((N_DEV, B_LOC, SQ, D), jnp.bfloat16),
            pltpu.VMEM((N_DEV, B_LOC, SQ, D), jnp.bfloat16),
            pltpu.VMEM((N_DEV - 1, B_LOC, SQ, D), jnp.bfloat16),
            pltpu.SemaphoreType.DMA((N_DEV - 1,)),
            pltpu.SemaphoreType.DMA((N_DEV - 1,)),
            pltpu.SemaphoreType.DMA((N_DEV - 1,)),
            pltpu.SemaphoreType.DMA((N_DEV - 1,)),
        ],
        compiler_params=pltpu.CompilerParams(collective_id=0),
    )(x, Wq, Wk, Wv, Wo)
